# Initial kernel scaffold; baseline (speedup 1.0000x reference)
import jax
import jax.numpy as jnp
from jax.experimental import pallas as pl


def kernel(queries, res2, res3, res4, res5, W_q, b_q, Wv2, bv2, Wv3, bv3, Wv4, bv4, Wv5, bv5, level_embed, W_value, b_value, W_off, b_off, W_attn, b_attn, W_ao, b_ao, W_out, b_out):
    raise NotImplementedError("write your pallas kernel here")


# speedup vs baseline: 4.5573x; 4.5573x over previous
import functools

import numpy as np
import jax
import jax.numpy as jnp
from jax import lax
from jax.experimental import pallas as pl
from jax.experimental.pallas import tpu as pltpu
from jax.experimental.pallas import tpu_sc as plsc

F32 = jnp.float32
I32 = jnp.int32

B = 4
NQ = 256
NH = 8
NL = 4
NP = 4
HID = 256
DH = 32
SIZES = (128, 64, 32, 16)
CDIMS = (128, 256, 512, 1024)
NW = 32
QPW = NQ // NW


def _fold_body(wv_ref, bias_ref, wval_ref, bval_ref, m_ref, c_ref):
    wval = wval_ref[...]
    m_ref[...] = jnp.dot(wv_ref[...], wval, preferred_element_type=F32)
    c_ref[...] = jnp.dot(bias_ref[...], wval, preferred_element_type=F32) + bval_ref[...]


def _fold_call(wv_cat, bias8, w_value, b_value):
    return pl.pallas_call(
        _fold_body,
        out_shape=(
            jax.ShapeDtypeStruct((sum(CDIMS), HID), F32),
            jax.ShapeDtypeStruct((8, HID), F32),
        ),
    )(wv_cat, bias8, w_value, b_value)


def _val_body(x_ref, m_ref, c_ref, o_ref):
    acc = lax.dot_general(x_ref[0], m_ref[...], (((0,), (0,)), ((), ())),
                          preferred_element_type=F32)
    o_ref[...] = (acc + c_ref[...])[None]


def _val_call(x, m, c, R):
    _, C, HW = x.shape
    return pl.pallas_call(
        _val_body,
        grid=(B, HW // R),
        in_specs=[
            pl.BlockSpec((1, C, R), lambda b, i: (b, 0, i)),
            pl.BlockSpec((C, HID), lambda b, i: (0, 0)),
            pl.BlockSpec((1, HID), lambda b, i: (0, 0)),
        ],
        out_specs=pl.BlockSpec((1, R, HID), lambda b, i: (b, i, 0)),
        out_shape=jax.ShapeDtypeStruct((B, HW, HID), F32),
    )(x, m, c)


def _prep_body(q_ref, wq_ref, bq_ref, wox_ref, box_ref, woy_ref, boy_ref,
               wat_ref, bat_ref, g_ref, cent_ref, bound_ref, sw_ref, hwl_ref,
               hcol_ref,
               i00_ref, i01_ref, i10_ref, i11_ref,
               w00_ref, w01_ref, w10_ref, w11_ref):
    b = pl.program_id(0)
    q = jnp.dot(q_ref[0], wq_ref[...], preferred_element_type=F32) + bq_ref[...]
    ox = jnp.dot(q, wox_ref[...], preferred_element_type=F32) + box_ref[...]
    oy = jnp.dot(q, woy_ref[...], preferred_element_type=F32) + boy_ref[...]
    logit = jnp.dot(q, wat_ref[...], preferred_element_type=F32) + bat_ref[...]
    m = jnp.max(logit, axis=1, keepdims=True)
    e = jnp.exp(logit - m)
    denom = jnp.dot(e, g_ref[...], preferred_element_type=F32)
    a = e / denom

    cent = cent_ref[...]
    bnd = bound_ref[...]
    ix = ox + cent
    iy = oy + cent
    x0 = jnp.floor(ix)
    y0 = jnp.floor(iy)
    fx = ix - x0
    fy = iy - y0
    one = jnp.float32(1.0)

    def valid(xf, yf):
        v = (xf >= 0) & (xf <= bnd) & (yf >= 0) & (yf <= bnd)
        return v.astype(F32)

    x1 = x0 + one
    y1 = y0 + one
    v00 = valid(x0, y0)
    v01 = valid(x1, y0)
    v10 = valid(x0, y1)
    v11 = valid(x1, y1)

    x0c = jnp.clip(x0, 0.0, bnd).astype(I32)
    x1c = jnp.clip(x1, 0.0, bnd).astype(I32)
    y0c = jnp.clip(y0, 0.0, bnd).astype(I32)
    y1c = jnp.clip(y1, 0.0, bnd).astype(I32)

    sw = sw_ref[...]
    hwl = hwl_ref[...]
    hcol = hcol_ref[...]

    def mkidx(yc, xc):
        return (b * hwl + yc * sw + xc) * NH + hcol

    i00 = mkidx(y0c, x0c)
    i01 = mkidx(y0c, x1c)
    i10 = mkidx(y1c, x0c)
    i11 = mkidx(y1c, x1c)

    w00 = a * (one - fy) * (one - fx) * v00
    w01 = a * (one - fy) * fx * v01
    w10 = a * fy * (one - fx) * v10
    w11 = a * fy * fx * v11

    for l in range(NL):
        sl = slice(l * 32, (l + 1) * 32)
        i00_ref[0, l] = i00[:, sl]
        i01_ref[0, l] = i01[:, sl]
        i10_ref[0, l] = i10[:, sl]
        i11_ref[0, l] = i11[:, sl]
        w00_ref[0, l] = w00[:, sl]
        w01_ref[0, l] = w01[:, sl]
        w10_ref[0, l] = w10[:, sl]
        w11_ref[0, l] = w11[:, sl]


def _prep_call(q3, wq, bq, wox, box, woy, boy, wat, bat, g, cent, bound, sw,
               hwl, hcol):
    full = lambda shape: pl.BlockSpec(shape, lambda b: tuple(0 for _ in shape))
    ospec = pl.BlockSpec((1, NL, NQ, 32), lambda b: (b, 0, 0, 0))
    oshape_i = jax.ShapeDtypeStruct((B, NL, NQ, 32), I32)
    oshape_w = jax.ShapeDtypeStruct((B, NL, NQ, 32), F32)
    return pl.pallas_call(
        _prep_body,
        grid=(B,),
        in_specs=[
            pl.BlockSpec((1, NQ, 2560), lambda b: (b, 0, 0)),
            full((2560, HID)), full((1, HID)),
            full((HID, 128)), full((1, 128)),
            full((HID, 128)), full((1, 128)),
            full((HID, 128)), full((1, 128)),
            full((128, 128)),
            full((1, 128)), full((1, 128)), full((1, 128)), full((1, 128)),
            full((1, 128)),
        ],
        out_specs=(ospec,) * 8,
        out_shape=(oshape_i,) * 4 + (oshape_w,) * 4,
    )(q3, wq, bq, wox, box, woy, boy, wat, bat, g, cent, bound, sw, hwl, hcol)


_GDN = lax.GatherDimensionNumbers(offset_dims=(), collapsed_slice_dims=(0,),
                                  start_index_map=(0,))


def _lane_bcast(vec, j):
    idx = jnp.full((16, 1), j, dtype=I32)
    return lax.gather(vec, idx, _GDN, (1,),
                      mode=lax.GatherScatterMode.PROMISE_IN_BOUNDS)


def _sc_body(t2, t3, t4, t5,
             i00, i01, i10, i11, w00, w01, w10, w11,
             out_ref,
             iv0, iv1, iv2, iv3, wv0, wv1, wv2r, wv3r,
             g0, g1, g2, g3, acc,
             s0, s1, s2, s3):
    tabs = (t2, t3, t4, t5)
    ihbm = (i00, i01, i10, i11)
    whbm = (w00, w01, w10, w11)
    ivs = (iv0, iv1, iv2, iv3)
    wvs = (wv0, wv1, wv2r, wv3r)
    gs = (g0, g1, g2, g3)
    sems = (s0, s1, s2, s3)

    wid = lax.axis_index("s") * 2 + lax.axis_index("c")
    q0 = wid * QPW

    def compute(c, b, store_first):
        gr = gs[c]
        wr = wvs[c]

        def kbody(k, _):
            wv16 = wr[pl.ds(k * 16, 16)]
            for j in range(16):
                row = k * 16 + j
                orow = b * 64 + k * 4 + (j // 4)
                wj = _lane_bcast(wv16, j)
                lo = wj * gr[row, pl.ds(0, 16)]
                hi = wj * gr[row, pl.ds(16, 16)]
                if store_first and (j % 4 == 0):
                    acc[orow, pl.ds(0, 16)] = lo
                    acc[orow, pl.ds(16, 16)] = hi
                else:
                    plsc.addupdate(acc.at[orow, pl.ds(0, 16)], lo)
                    plsc.addupdate(acc.at[orow, pl.ds(16, 16)], hi)
            return _

        lax.fori_loop(0, 16, kbody, None)

    for l in range(NL):
        tab = tabs[l]

        def bbody(b, _, l=l, tab=tab):
            for c in range(4):
                pltpu.sync_copy(ihbm[c].at[b, l, pl.ds(q0 * 32, 256)], ivs[c])
                pltpu.sync_copy(whbm[c].at[b, l, pl.ds(q0 * 32, 256)], wvs[c])
            handles = [pltpu.async_copy(tab.at[ivs[c]], gs[c], sems[c])
                       for c in range(4)]
            for c in range(4):
                handles[c].wait()
                compute(c, b, store_first=(l == 0 and c == 0))
            return _

        lax.fori_loop(0, B, bbody, None)

    def wout(b, _):
        pltpu.sync_copy(acc.at[pl.ds(b * 64, 64)],
                        out_ref.at[pl.ds(b * 2048 + q0 * 8, 64)])
        return _

    lax.fori_loop(0, B, wout, None)


def _sc_gather(tabs, idxs, ws):
    mesh = plsc.VectorSubcoreMesh(core_axis_name="c", subcore_axis_name="s")
    kern = pl.kernel(
        _sc_body,
        out_type=jax.ShapeDtypeStruct((B * NQ * NH, DH), F32),
        mesh=mesh,
        compiler_params=pltpu.CompilerParams(use_tc_tiling_on_sc=False),
        scratch_types=(
            [pltpu.VMEM((256,), I32) for _ in range(4)]
            + [pltpu.VMEM((256,), F32) for _ in range(4)]
            + [pltpu.VMEM((256, DH), F32) for _ in range(4)]
            + [pltpu.VMEM((B * 64, DH), F32)]
            + [pltpu.SemaphoreType.DMA for _ in range(4)]
        ),
    )
    return kern(*tabs, *idxs, *ws)


def _out_body(x_ref, wao_ref, bao_ref, wout_ref, bout_ref, o_ref):
    t = jnp.dot(x_ref[...], wao_ref[...], preferred_element_type=F32) + bao_ref[...]
    o_ref[...] = jnp.dot(t, wout_ref[...], preferred_element_type=F32) + bout_ref[...]


def _out_call(x, wao, bao, wout, bout):
    return pl.pallas_call(
        _out_body,
        out_shape=jax.ShapeDtypeStruct((B * NQ, 2560), F32),
    )(x, wao, bao, wout, bout)


_COLS = np.arange(128)
_L_OF = _COLS // 32
_H_OF = (_COLS % 32) // 4
_P_OF = _COLS % 4
_PERM_ATTN = _H_OF * 16 + _L_OF * 4 + _P_OF
_PERM_OFF_X = ((_H_OF * 4 + _L_OF) * 4 + _P_OF) * 2
_PERM_OFF_Y = _PERM_OFF_X + 1
_G_NP = (_H_OF[:, None] == _H_OF[None, :]).astype(np.float32)
_S_NP = np.array(SIZES, np.float32)[_L_OF]
_CENT_NP = (_S_NP / 2.0 - 0.5).astype(np.float32)[None]
_BOUND_NP = (_S_NP - 1.0).astype(np.float32)[None]
_SW_NP = _S_NP.astype(np.int32)[None]
_HWL_NP = (_S_NP * _S_NP).astype(np.int32)[None]
_HCOL_NP = _H_OF.astype(np.int32)[None]


def kernel(queries, res2, res3, res4, res5, W_q, b_q, Wv2, bv2, Wv3, bv3,
           Wv4, bv4, Wv5, bv5, level_embed, W_value, b_value, W_off, b_off,
           W_attn, b_attn, W_ao, b_ao, W_out, b_out):
    wv_cat = jnp.concatenate([Wv2, Wv3, Wv4, Wv5], axis=0)
    bias4 = jnp.stack([bv2, bv3, bv4, bv5]) + level_embed
    bias8 = jnp.concatenate([bias4, jnp.zeros((4, HID), F32)], axis=0)
    m_cat, c8 = _fold_call(wv_cat, bias8, W_value, b_value.reshape(1, HID))

    feats = (res2, res3, res4, res5)
    r_blocks = (2048, 1024, 1024, 256)
    tabs = []
    start = 0
    for l in range(NL):
        C, S = CDIMS[l], SIZES[l]
        m_l = lax.slice(m_cat, (start, 0), (start + C, HID))
        c_l = lax.slice(c8, (l, 0), (l + 1, HID))
        x = feats[l].reshape(B, C, S * S)
        v = _val_call(x, m_l, c_l, r_blocks[l])
        tabs.append(v.reshape(B * S * S * NH, DH))
        start += C

    wox = W_off[:, _PERM_OFF_X]
    box = b_off[_PERM_OFF_X].reshape(1, 128)
    woy = W_off[:, _PERM_OFF_Y]
    boy = b_off[_PERM_OFF_Y].reshape(1, 128)
    wat = W_attn[:, _PERM_ATTN]
    bat = b_attn[_PERM_ATTN].reshape(1, 128)
    outs = _prep_call(queries, W_q, b_q.reshape(1, HID), wox, box, woy, boy,
                      wat, bat, jnp.asarray(_G_NP), jnp.asarray(_CENT_NP),
                      jnp.asarray(_BOUND_NP), jnp.asarray(_SW_NP),
                      jnp.asarray(_HWL_NP), jnp.asarray(_HCOL_NP))
    idxs = [o.reshape(B, NL, NQ * 32) for o in outs[:4]]
    ws = [o.reshape(B, NL, NQ * 32) for o in outs[4:]]

    sampled = _sc_gather(tabs, idxs, ws)

    out = _out_call(sampled.reshape(B * NQ, HID), W_ao, b_ao.reshape(1, HID),
                    W_out, b_out.reshape(1, 2560))
    return out.reshape(B, NQ, 2560)


# speedup vs baseline: 5.5852x; 1.2255x over previous
import functools

import numpy as np
import jax
import jax.numpy as jnp
from jax import lax
from jax.experimental import pallas as pl
from jax.experimental.pallas import tpu as pltpu
from jax.experimental.pallas import tpu_sc as plsc

F32 = jnp.float32
I32 = jnp.int32

B = 4
NQ = 256
NH = 8
NL = 4
NP = 4
HID = 256
DH = 32
SIZES = (128, 64, 32, 16)
CDIMS = (128, 256, 512, 1024)
NW = 32
QPW = NQ // NW


def _fold_body(wv_ref, bias_ref, wval_ref, bval_ref, m_ref, c_ref):
    wval = wval_ref[...]
    m_ref[...] = jnp.dot(wv_ref[...], wval, preferred_element_type=F32)
    c_ref[...] = jnp.dot(bias_ref[...], wval, preferred_element_type=F32) + bval_ref[...]


def _fold_call(wv_cat, bias8, w_value, b_value):
    return pl.pallas_call(
        _fold_body,
        out_shape=(
            jax.ShapeDtypeStruct((sum(CDIMS), HID), F32),
            jax.ShapeDtypeStruct((8, HID), F32),
        ),
    )(wv_cat, bias8, w_value, b_value)


def _val_body(x_ref, m_ref, c_ref, o_ref):
    acc = lax.dot_general(x_ref[0], m_ref[...], (((0,), (0,)), ((), ())),
                          preferred_element_type=F32) + c_ref[...]
    R = acc.shape[0]
    o_ref[0, :, 0] = acc[:, :128].reshape(R // 8, 8, 128)
    o_ref[0, :, 1] = acc[:, 128:].reshape(R // 8, 8, 128)


def _val_call(x, m, c, R):
    _, C, HW = x.shape
    return pl.pallas_call(
        _val_body,
        grid=(B, HW // R),
        in_specs=[
            pl.BlockSpec((1, C, R), lambda b, i: (b, 0, i)),
            pl.BlockSpec((C, HID), lambda b, i: (0, 0)),
            pl.BlockSpec((1, HID), lambda b, i: (0, 0)),
        ],
        out_specs=pl.BlockSpec((1, R // 8, 2, 8, 128), lambda b, i: (b, i, 0, 0, 0)),
        out_shape=jax.ShapeDtypeStruct((B, HW // 8, 2, 8, 128), F32),
    )(x, m, c)


def _prep_body(q_ref, wq_ref, bq_ref, wox_ref, box_ref, woy_ref, boy_ref,
               wat_ref, bat_ref, g_ref, cent_ref, bound_ref, sw_ref, hw8_ref,
               hdiv_ref, hmod_ref, i_ref, w_ref):
    b = pl.program_id(0)
    q = jnp.dot(q_ref[0], wq_ref[...], preferred_element_type=F32) + bq_ref[...]
    ox = jnp.dot(q, wox_ref[...], preferred_element_type=F32) + box_ref[...]
    oy = jnp.dot(q, woy_ref[...], preferred_element_type=F32) + boy_ref[...]
    logit = jnp.dot(q, wat_ref[...], preferred_element_type=F32) + bat_ref[...]
    m = jnp.max(logit, axis=1, keepdims=True)
    e = jnp.exp(logit - m)
    denom = jnp.dot(e, g_ref[...], preferred_element_type=F32)
    a = e / denom

    cent = cent_ref[...]
    bnd = bound_ref[...]
    ix = ox + cent
    iy = oy + cent
    x0 = jnp.floor(ix)
    y0 = jnp.floor(iy)
    fx = ix - x0
    fy = iy - y0
    one = jnp.float32(1.0)

    def valid(xf, yf):
        v = (xf >= 0) & (xf <= bnd) & (yf >= 0) & (yf <= bnd)
        return v.astype(F32)

    x1 = x0 + one
    y1 = y0 + one
    v00 = valid(x0, y0)
    v01 = valid(x1, y0)
    v10 = valid(x0, y1)
    v11 = valid(x1, y1)

    x0c = jnp.clip(x0, 0.0, bnd).astype(I32)
    x1c = jnp.clip(x1, 0.0, bnd).astype(I32)
    y0c = jnp.clip(y0, 0.0, bnd).astype(I32)
    y1c = jnp.clip(y1, 0.0, bnd).astype(I32)

    sw = sw_ref[...]
    hw8 = hw8_ref[...]
    hdiv = hdiv_ref[...]
    hmod = hmod_ref[...]

    def mkidx(yc, xc):
        pos = yc * sw + xc
        p_maj = jnp.right_shift(pos, 3)
        s_min = jnp.bitwise_and(pos, 7)
        return (b * hw8 + p_maj) * 64 + hdiv * 32 + s_min * 4 + hmod

    idxs = (mkidx(y0c, x0c), mkidx(y0c, x1c), mkidx(y1c, x0c), mkidx(y1c, x1c))
    wts = (a * (one - fy) * (one - fx) * v00,
           a * (one - fy) * fx * v01,
           a * fy * (one - fx) * v10,
           a * fy * fx * v11)

    for l in range(NL):
        sl = slice(l * 32, (l + 1) * 32)
        for c in range(4):
            i_ref[c, 0, l] = idxs[c][:, sl]
            w_ref[c, 0, l] = wts[c][:, sl]


def _prep_call(q3, wq, bq, wox, box, woy, boy, wat, bat, g, cent, bound, sw,
               hw8, hdiv, hmod):
    full = lambda shape: pl.BlockSpec(shape, lambda b: tuple(0 for _ in shape))
    ospec = pl.BlockSpec((4, 1, NL, NQ, 32), lambda b: (0, b, 0, 0, 0))
    return pl.pallas_call(
        _prep_body,
        grid=(B,),
        in_specs=[
            pl.BlockSpec((1, NQ, 2560), lambda b: (b, 0, 0)),
            full((2560, HID)), full((1, HID)),
            full((HID, 128)), full((1, 128)),
            full((HID, 128)), full((1, 128)),
            full((HID, 128)), full((1, 128)),
            full((128, 128)),
            full((1, 128)), full((1, 128)), full((1, 128)), full((1, 128)),
            full((1, 128)), full((1, 128)),
        ],
        out_specs=(ospec, ospec),
        out_shape=(jax.ShapeDtypeStruct((4, B, NL, NQ, 32), I32),
                   jax.ShapeDtypeStruct((4, B, NL, NQ, 32), F32)),
    )(q3, wq, bq, wox, box, woy, boy, wat, bat, g, cent, bound, sw, hw8,
      hdiv, hmod)


_GDN = lax.GatherDimensionNumbers(offset_dims=(), collapsed_slice_dims=(0,),
                                  start_index_map=(0,))


def _lane_bcast(vec, j):
    idx = jnp.full((16, 1), j, dtype=I32)
    return lax.gather(vec, idx, _GDN, (1,),
                      mode=lax.GatherScatterMode.PROMISE_IN_BOUNDS)


_NSTEP = NL * B


def _sc_body(t2, t3, t4, t5, ihbm, whbm, out_ref,
             ivs, wvs, g, acc,
             si0, si1, si2, sw0, sw1, sw2,
             sg00, sg01, sg02, sg03, sg10, sg11, sg12, sg13):
    tabs = (t2, t3, t4, t5)
    si = (si0, si1, si2)
    sw = (sw0, sw1, sw2)
    sg = ((sg00, sg01, sg02, sg03), (sg10, sg11, sg12, sg13))

    wid = lax.axis_index("s") * 2 + lax.axis_index("c")
    q0m32 = wid * (QPW * 32)

    def stage(t):
        ss = t % 3
        b, l = t % 4, t // 4
        pltpu.async_copy(ihbm.at[:, b, l, pl.ds(q0m32, 256)], ivs.at[ss], si[ss])
        pltpu.async_copy(whbm.at[:, b, l, pl.ds(q0m32, 256)], wvs.at[ss], sw[ss])

    def fire(t):
        ss, gb, l = t % 3, t % 2, t // 4
        pltpu.make_async_copy(ihbm.at[:, 0, 0, pl.ds(0, 256)], ivs.at[ss],
                              si[ss]).wait()
        pltpu.make_async_copy(whbm.at[:, 0, 0, pl.ds(0, 256)], wvs.at[ss],
                              sw[ss]).wait()
        for c in range(4):
            pltpu.async_copy(tabs[l].at[ivs.at[ss, c]], g.at[gb, c], sg[gb][c])

    def compute(t):
        ss, gb, l, b = t % 3, t % 2, t // 4, t % 4
        for c in range(4):
            pltpu.make_async_copy(tabs[0].at[pl.ds(0, 256)], g.at[gb, c],
                                  sg[gb][c]).wait()

        def kb(k, _):
            wvecs = [wvs[ss, c, pl.ds(k * 16, 16)] for c in range(4)]
            for rr in range(4):
                arow = b * 64 + k * 4 + rr
                if l == 0:
                    al = jnp.zeros((16,), F32)
                    ah = jnp.zeros((16,), F32)
                else:
                    al = acc[arow, pl.ds(0, 16)]
                    ah = acc[arow, pl.ds(16, 16)]
                for c in range(4):
                    for p in range(4):
                        j = rr * 4 + p
                        wj = _lane_bcast(wvecs[c], j)
                        al = al + wj * g[gb, c, k * 16 + j, pl.ds(0, 16)]
                        ah = ah + wj * g[gb, c, k * 16 + j, pl.ds(16, 16)]
                acc[arow, pl.ds(0, 16)] = al
                acc[arow, pl.ds(16, 16)] = ah
            return _

        lax.fori_loop(0, 16, kb, None)

    for t in range(_NSTEP + 2):
        if 1 <= t <= _NSTEP:
            fire(t - 1)
        if t < _NSTEP:
            stage(t)
        if t >= 2:
            compute(t - 2)

    for b in range(B):
        pltpu.sync_copy(acc.at[pl.ds(b * 64, 64)],
                        out_ref.at[pl.ds(b * 2048 + wid * 64, 64)])


def _sc_gather(tabs, ihbm, whbm):
    mesh = plsc.VectorSubcoreMesh(core_axis_name="c", subcore_axis_name="s")
    kern = pl.kernel(
        _sc_body,
        out_type=jax.ShapeDtypeStruct((B * NQ * NH, DH), F32),
        mesh=mesh,
        compiler_params=pltpu.CompilerParams(use_tc_tiling_on_sc=False),
        scratch_types=(
            [pltpu.VMEM((3, 4, 256), I32),
             pltpu.VMEM((3, 4, 256), F32),
             pltpu.VMEM((2, 4, 256, DH), F32),
             pltpu.VMEM((B * 64, DH), F32)]
            + [pltpu.SemaphoreType.DMA for _ in range(14)]
        ),
    )
    return kern(*tabs, ihbm, whbm)


def _out_body(x_ref, wao_ref, bao_ref, wout_ref, bout_ref, o_ref):
    t = jnp.dot(x_ref[...], wao_ref[...], preferred_element_type=F32) + bao_ref[...]
    o_ref[...] = jnp.dot(t, wout_ref[...], preferred_element_type=F32) + bout_ref[...]


def _out_call(x, wao, bao, wout, bout):
    return pl.pallas_call(
        _out_body,
        out_shape=jax.ShapeDtypeStruct((B * NQ, 2560), F32),
    )(x, wao, bao, wout, bout)


_COLS = np.arange(128)
_L_OF = _COLS // 32
_H_OF = (_COLS % 32) // 4
_P_OF = _COLS % 4
_PERM_ATTN = _H_OF * 16 + _L_OF * 4 + _P_OF
_PERM_OFF_X = ((_H_OF * 4 + _L_OF) * 4 + _P_OF) * 2
_PERM_OFF_Y = _PERM_OFF_X + 1
_G_NP = (_H_OF[:, None] == _H_OF[None, :]).astype(np.float32)
_S_NP = np.array(SIZES, np.float32)[_L_OF]
_CENT_NP = (_S_NP / 2.0 - 0.5).astype(np.float32)[None]
_BOUND_NP = (_S_NP - 1.0).astype(np.float32)[None]
_SW_NP = _S_NP.astype(np.int32)[None]
_HW8_NP = (_S_NP * _S_NP / 8).astype(np.int32)[None]
_HDIV_NP = (_H_OF // 4).astype(np.int32)[None]
_HMOD_NP = (_H_OF % 4).astype(np.int32)[None]


def kernel(queries, res2, res3, res4, res5, W_q, b_q, Wv2, bv2, Wv3, bv3,
           Wv4, bv4, Wv5, bv5, level_embed, W_value, b_value, W_off, b_off,
           W_attn, b_attn, W_ao, b_ao, W_out, b_out):
    wv_cat = jnp.concatenate([Wv2, Wv3, Wv4, Wv5], axis=0)
    bias4 = jnp.stack([bv2, bv3, bv4, bv5]) + level_embed
    bias8 = jnp.concatenate([bias4, jnp.zeros((4, HID), F32)], axis=0)
    m_cat, c8 = _fold_call(wv_cat, bias8, W_value, b_value.reshape(1, HID))

    feats = (res2, res3, res4, res5)
    r_blocks = (2048, 1024, 1024, 256)
    tabs = []
    start = 0
    for l in range(NL):
        C, S = CDIMS[l], SIZES[l]
        m_l = lax.slice(m_cat, (start, 0), (start + C, HID))
        c_l = lax.slice(c8, (l, 0), (l + 1, HID))
        x = feats[l].reshape(B, C, S * S)
        v5 = _val_call(x, m_l, c_l, r_blocks[l])
        tabs.append(v5.reshape(B * S * S * NH, DH))
        start += C

    wox = W_off[:, _PERM_OFF_X]
    box = b_off[_PERM_OFF_X].reshape(1, 128)
    woy = W_off[:, _PERM_OFF_Y]
    boy = b_off[_PERM_OFF_Y].reshape(1, 128)
    wat = W_attn[:, _PERM_ATTN]
    bat = b_attn[_PERM_ATTN].reshape(1, 128)
    idx_arr, w_arr = _prep_call(
        queries, W_q, b_q.reshape(1, HID), wox, box, woy, boy,
        wat, bat, jnp.asarray(_G_NP), jnp.asarray(_CENT_NP),
        jnp.asarray(_BOUND_NP), jnp.asarray(_SW_NP),
        jnp.asarray(_HW8_NP), jnp.asarray(_HDIV_NP), jnp.asarray(_HMOD_NP))

    sampled = _sc_gather(tabs, idx_arr.reshape(4, B, NL, NQ * 32),
                         w_arr.reshape(4, B, NL, NQ * 32))

    out = _out_call(sampled.reshape(B * NQ, HID), W_ao, b_ao.reshape(1, HID),
                    W_out, b_out.reshape(1, 2560))
    return out.reshape(B, NQ, 2560)
